# baseline (speedup 1.0000x reference)
import functools

import jax
import jax.numpy as jnp
from jax import lax
from jax.experimental import pallas as pl
from jax.experimental.pallas import tpu as pltpu
from jax.experimental.pallas import tpu_sc as plsc

B, N, D = 4096, 100, 64
K = 50

_info = plsc.get_sparse_core_info()
_NC, _NS = _info.num_cores, _info.num_subcores
_NW = _NC * _NS
_BPW = B // _NW

_mesh = plsc.VectorSubcoreMesh(core_axis_name="c", subcore_axis_name="s")


@functools.partial(
    pl.kernel,
    mesh=_mesh,
    out_type=jax.ShapeDtypeStruct((B, K, 1, D), jnp.float32),
)
def _sc_index_select(x_hbm, out_hbm):
    wid = lax.axis_index("s") * _NC + lax.axis_index("c")
    base = wid * _BPW
    pltpu.sync_copy(
        x_hbm.at[pl.ds(base, _BPW), :, pl.ds(0, 1), :],
        out_hbm.at[pl.ds(base, _BPW)],
    )


def kernel(x):
    x4 = x.reshape(B, K, 2, D)
    out = _sc_index_select(x4)
    return out.reshape(B, K, D)


# speedup vs baseline: 6.5308x; 6.5308x over previous
import functools

import jax
import jax.numpy as jnp
from jax import lax
from jax.experimental import pallas as pl
from jax.experimental.pallas import tpu as pltpu
from jax.experimental.pallas import tpu_sc as plsc

B, N, D = 4096, 100, 64
K = 50

_info = plsc.get_sparse_core_info()
_NC, _NS = _info.num_cores, _info.num_subcores
_NW = _NC * _NS
_BPW = B // _NW
_NB = 8
_CHUNKS = _BPW // _NB

_mesh = plsc.VectorSubcoreMesh(core_axis_name="c", subcore_axis_name="s")


@functools.partial(
    pl.kernel,
    mesh=_mesh,
    out_type=jax.ShapeDtypeStruct((B, K, 1, D), jnp.float32),
    scratch_types=[
        pltpu.VMEM((2, _NB, K, 1, D), jnp.float32),
        pltpu.SemaphoreType.DMA,
        pltpu.SemaphoreType.DMA,
        pltpu.SemaphoreType.DMA,
        pltpu.SemaphoreType.DMA,
    ],
)
def _sc_index_select(x_hbm, out_hbm, buf, sin0, sin1, sout0, sout1):
    wid = lax.axis_index("s") * _NC + lax.axis_index("c")
    base = wid * _BPW
    sins = (sin0, sin1)
    souts = (sout0, sout1)

    def start_in(i):
        slot = i % 2
        return pltpu.async_copy(
            x_hbm.at[pl.ds(base + i * _NB, _NB), :, pl.ds(0, 1), :],
            buf.at[slot],
            sins[slot],
        )

    def start_out(i):
        slot = i % 2
        return pltpu.async_copy(
            buf.at[slot],
            out_hbm.at[pl.ds(base + i * _NB, _NB)],
            souts[slot],
        )

    ins, outs = {}, {}
    ins[0] = start_in(0)
    for i in range(_CHUNKS):
        if i + 1 < _CHUNKS:
            if i - 1 >= 0:
                outs[i - 1].wait()
            ins[i + 1] = start_in(i + 1)
        ins[i].wait()
        outs[i] = start_out(i)
    outs[_CHUNKS - 2].wait()
    outs[_CHUNKS - 1].wait()


def kernel(x):
    x4 = x.reshape(B, K, 2, D)
    out = _sc_index_select(x4)
    return out.reshape(B, K, D)
